# baseline (speedup 1.0000x reference)
import functools

import jax
import jax.numpy as jnp
from jax import lax
from jax.experimental import pallas as pl
from jax.experimental.pallas import tpu as pltpu
from jax.experimental.pallas import tpu_sc as plsc

_N = 1048576
_NC = 2
_NS = 16
_NW = _NC * _NS
_L = 16
_R = _N // _NW
_CHUNK = 8192
_NCHUNK = _R // _CHUNK
_ITERS = _CHUNK // _L
_UNROLL = 4


def _sc_body(o_hbm, t_hbm, m_hbm, out_hbm,
             ob0, ob1, tb0, tb1, mb0, mb1, accv, sem0, sem1):
    wid = lax.axis_index("s") * _NC + lax.axis_index("c")
    iota2 = lax.iota(jnp.int32, _L) * 2

    bufs = ((ob0, tb0, mb0, sem0), (ob1, tb1, mb1, sem1))

    def start_chunk(g, buf):
        ob, tb, mb, sem = buf
        base = wid * _R + g * _CHUNK
        h1 = pltpu.async_copy(o_hbm.at[pl.ds(base * 2, _CHUNK * 2)], ob, sem)
        h2 = pltpu.async_copy(t_hbm.at[pl.ds(base, _CHUNK)], tb, sem)
        h3 = pltpu.async_copy(m_hbm.at[pl.ds(base, _CHUNK)], mb, sem)
        return (h1, h2, h3)

    def do_chunk(buf, acc):
        ob, tb, mb, _ = buf

        def group(j, u, aa, at):
            off = (j + u) * _L
            t = tb[pl.ds(off, _L)]
            mk = mb[pl.ds(off, _L)]
            idx = iota2 + ((j + u) * (2 * _L)) + t
            a = plsc.load_gather(ob, [idx])
            b = plsc.load_gather(ob, [idx ^ 1])
            z = b - a
            mx = jnp.maximum(z, 0.0)
            e = jnp.exp(-jnp.abs(z))
            p = -0.023979573
            p = p * e + 0.10150005
            p = p * e + -0.21029369
            p = p * e + 0.32529514
            p = p * e + -0.4993726
            p = p * e + 0.99999183
            sp = mx + e * p
            return aa + sp, at + mk.astype(jnp.float32) * sp

        def body(j, carry):
            accs = list(carry)
            for u in range(_UNROLL):
                aa, at = accs[u]
                accs[u] = group(j, u, aa, at)
            return tuple(accs)

        return lax.fori_loop(0, _ITERS // _UNROLL,
                             lambda j, c: body(j * _UNROLL, c), acc)

    zeros = jnp.zeros((_L,), jnp.float32)
    acc = tuple((zeros, zeros) for _ in range(_UNROLL))
    handles = start_chunk(0, bufs[0])
    for g in range(_NCHUNK):
        nxt = start_chunk(g + 1, bufs[(g + 1) % 2]) if g + 1 < _NCHUNK else None
        for h in handles:
            h.wait()
        acc = do_chunk(bufs[g % 2], acc)
        handles = nxt

    acc_all = acc[0][0]
    acc_thin = acc[0][1]
    for u in range(1, _UNROLL):
        acc_all = acc_all + acc[u][0]
        acc_thin = acc_thin + acc[u][1]
    accv[pl.ds(0, _L)] = acc_all
    accv[pl.ds(_L, _L)] = acc_thin
    pltpu.sync_copy(accv, out_hbm.at[pl.ds(wid * (2 * _L), 2 * _L)])


_sc_kernel = functools.partial(
    pl.kernel,
    mesh=plsc.VectorSubcoreMesh(core_axis_name="c", subcore_axis_name="s"),
    out_type=jax.ShapeDtypeStruct((_NW * 2 * _L,), jnp.float32),
    scratch_types=[
        pltpu.VMEM((_CHUNK * 2,), jnp.float32),
        pltpu.VMEM((_CHUNK * 2,), jnp.float32),
        pltpu.VMEM((_CHUNK,), jnp.int32),
        pltpu.VMEM((_CHUNK,), jnp.int32),
        pltpu.VMEM((_CHUNK,), jnp.int32),
        pltpu.VMEM((_CHUNK,), jnp.int32),
        pltpu.VMEM((2 * _L,), jnp.float32),
        pltpu.SemaphoreType.DMA,
        pltpu.SemaphoreType.DMA,
    ],
    compiler_params=pltpu.CompilerParams(needs_layout_passes=False),
)(_sc_body)


def kernel(outputs, targets, thin_mask, thin_weight):
    o_flat = outputs.reshape(-1)
    partials = _sc_kernel(o_flat, targets, thin_mask)
    pr = partials.reshape(_NW, 2, _L)
    s_all = jnp.sum(pr[:, 0, :])
    s_thin = jnp.sum(pr[:, 1, :])
    tw = jnp.asarray(thin_weight, jnp.float32)
    loss = (s_all + (tw - 1.0) * s_thin) * (1.0 / _N)
    return loss.astype(jnp.float32)


# speedup vs baseline: 30.1231x; 30.1231x over previous
import functools

import jax
import jax.numpy as jnp
from jax import lax
from jax.experimental import pallas as pl
from jax.experimental.pallas import tpu as pltpu
from jax.experimental.pallas import tpu_sc as plsc

_N = 1048576
_NC = 2
_NS = 16
_NW = _NC * _NS
_L = 16
_R = _N // _NW
_CHUNK = 8192
_NCHUNK = _R // _CHUNK
_ITERS = _CHUNK // _L
_UNROLL = 4


def _sc_body(o0_hbm, o1_hbm, t_hbm, m_hbm, out_hbm,
             b0, b1, b2, b3, b4, b5, b6, b7, accv, sem0, sem1):
    wid = lax.axis_index("s") * _NC + lax.axis_index("c")

    bufs = ((b0, b1, b2, b3, sem0), (b4, b5, b6, b7, sem1))

    def start_chunk(g, buf):
        ob0, ob1, tb, mb, sem = buf
        base = wid * _R + g * _CHUNK
        sl = pl.ds(base, _CHUNK)
        return (pltpu.async_copy(o0_hbm.at[sl], ob0, sem),
                pltpu.async_copy(o1_hbm.at[sl], ob1, sem),
                pltpu.async_copy(t_hbm.at[sl], tb, sem),
                pltpu.async_copy(m_hbm.at[sl], mb, sem))

    def do_chunk(buf, acc):
        ob0, ob1, tb, mb, _ = buf

        def group(j, u, aa, at):
            sl = pl.ds((j + u) * _L, _L)
            v0 = ob0[sl]
            v1 = ob1[sl]
            t = tb[sl]
            mk = mb[sl]
            d = v1 - v0
            sf = (1 - 2 * t).astype(jnp.float32)
            mx = jnp.maximum(sf * d, 0.0)
            e = jnp.exp(-jnp.abs(d))
            p = -0.023979573
            p = p * e + 0.10150005
            p = p * e + -0.21029369
            p = p * e + 0.32529514
            p = p * e + -0.4993726
            p = p * e + 0.99999183
            sp = mx + e * p
            return aa + sp, at + mk.astype(jnp.float32) * sp

        def body(j, carry):
            accs = list(carry)
            for u in range(_UNROLL):
                aa, at = accs[u]
                accs[u] = group(j, u, aa, at)
            return tuple(accs)

        return lax.fori_loop(0, _ITERS // _UNROLL,
                             lambda j, c: body(j * _UNROLL, c), acc)

    zeros = jnp.zeros((_L,), jnp.float32)
    acc = tuple((zeros, zeros) for _ in range(_UNROLL))
    handles = start_chunk(0, bufs[0])
    for g in range(_NCHUNK):
        nxt = start_chunk(g + 1, bufs[(g + 1) % 2]) if g + 1 < _NCHUNK else None
        for h in handles:
            h.wait()
        acc = do_chunk(bufs[g % 2], acc)
        handles = nxt

    acc_all = acc[0][0]
    acc_thin = acc[0][1]
    for u in range(1, _UNROLL):
        acc_all = acc_all + acc[u][0]
        acc_thin = acc_thin + acc[u][1]
    accv[pl.ds(0, _L)] = acc_all
    accv[pl.ds(_L, _L)] = acc_thin
    pltpu.sync_copy(accv, out_hbm.at[pl.ds(wid * (2 * _L), 2 * _L)])


_sc_kernel = functools.partial(
    pl.kernel,
    mesh=plsc.VectorSubcoreMesh(core_axis_name="c", subcore_axis_name="s"),
    out_type=jax.ShapeDtypeStruct((_NW * 2 * _L,), jnp.float32),
    scratch_types=[
        pltpu.VMEM((_CHUNK,), jnp.float32),
        pltpu.VMEM((_CHUNK,), jnp.float32),
        pltpu.VMEM((_CHUNK,), jnp.int32),
        pltpu.VMEM((_CHUNK,), jnp.int32),
        pltpu.VMEM((_CHUNK,), jnp.float32),
        pltpu.VMEM((_CHUNK,), jnp.float32),
        pltpu.VMEM((_CHUNK,), jnp.int32),
        pltpu.VMEM((_CHUNK,), jnp.int32),
        pltpu.VMEM((2 * _L,), jnp.float32),
        pltpu.SemaphoreType.DMA,
        pltpu.SemaphoreType.DMA,
    ],
    compiler_params=pltpu.CompilerParams(needs_layout_passes=False),
)(_sc_body)


def kernel(outputs, targets, thin_mask, thin_weight):
    o0 = outputs[:, 0]
    o1 = outputs[:, 1]
    partials = _sc_kernel(o0, o1, targets, thin_mask)
    pr = partials.reshape(_NW, 2, _L)
    s_all = jnp.sum(pr[:, 0, :])
    s_thin = jnp.sum(pr[:, 1, :])
    tw = jnp.asarray(thin_weight, jnp.float32)
    loss = (s_all + (tw - 1.0) * s_thin) * (1.0 / _N)
    return loss.astype(jnp.float32)


# speedup vs baseline: 32.4209x; 1.0763x over previous
import functools

import jax
import jax.numpy as jnp
from jax import lax
from jax.experimental import pallas as pl
from jax.experimental.pallas import tpu as pltpu
from jax.experimental.pallas import tpu_sc as plsc

_N = 1048576
_NC = 2
_NS = 16
_NW = _NC * _NS
_L = 16
_R = _N // _NW
_CHUNK = 8192
_NCHUNK = _R // _CHUNK
_ITERS = _CHUNK // _L
_UNROLL = 4


def _sc_body(d_hbm, t_hbm, m_hbm, out_hbm,
             b0, b1, b2, b3, b4, b5, accv, sem0, sem1):
    wid = lax.axis_index("s") * _NC + lax.axis_index("c")

    bufs = ((b0, b1, b2, sem0), (b3, b4, b5, sem1))

    def start_chunk(g, buf):
        db, tb, mb, sem = buf
        base = wid * _R + g * _CHUNK
        sl = pl.ds(base, _CHUNK)
        return (pltpu.async_copy(d_hbm.at[sl], db, sem),
                pltpu.async_copy(t_hbm.at[sl], tb, sem),
                pltpu.async_copy(m_hbm.at[sl], mb, sem))

    def do_chunk(buf, acc):
        db, tb, mb, _ = buf

        def group(j, u, aa, at):
            sl = pl.ds((j + u) * _L, _L)
            dv = db[sl]
            t = tb[sl]
            mk = mb[sl]
            sf = (1 - 2 * t).astype(jnp.float32)
            mx = jnp.maximum(sf * dv, 0.0)
            e = jnp.exp(-jnp.abs(dv))
            p = -0.023979573
            p = p * e + 0.10150005
            p = p * e + -0.21029369
            p = p * e + 0.32529514
            p = p * e + -0.4993726
            p = p * e + 0.99999183
            sp = mx + e * p
            return aa + sp, at + mk.astype(jnp.float32) * sp

        def body(j, carry):
            accs = list(carry)
            for u in range(_UNROLL):
                aa, at = accs[u]
                accs[u] = group(j, u, aa, at)
            return tuple(accs)

        return lax.fori_loop(0, _ITERS // _UNROLL,
                             lambda j, c: body(j * _UNROLL, c), acc)

    zeros = jnp.zeros((_L,), jnp.float32)
    acc = tuple((zeros, zeros) for _ in range(_UNROLL))
    handles = start_chunk(0, bufs[0])
    for g in range(_NCHUNK):
        nxt = start_chunk(g + 1, bufs[(g + 1) % 2]) if g + 1 < _NCHUNK else None
        for h in handles:
            h.wait()
        acc = do_chunk(bufs[g % 2], acc)
        handles = nxt

    acc_all = acc[0][0]
    acc_thin = acc[0][1]
    for u in range(1, _UNROLL):
        acc_all = acc_all + acc[u][0]
        acc_thin = acc_thin + acc[u][1]
    accv[pl.ds(0, _L)] = acc_all
    accv[pl.ds(_L, _L)] = acc_thin
    pltpu.sync_copy(accv, out_hbm.at[pl.ds(wid * (2 * _L), 2 * _L)])


_sc_kernel = functools.partial(
    pl.kernel,
    mesh=plsc.VectorSubcoreMesh(core_axis_name="c", subcore_axis_name="s"),
    out_type=jax.ShapeDtypeStruct((_NW * 2 * _L,), jnp.float32),
    scratch_types=[
        pltpu.VMEM((_CHUNK,), jnp.float32),
        pltpu.VMEM((_CHUNK,), jnp.int32),
        pltpu.VMEM((_CHUNK,), jnp.int32),
        pltpu.VMEM((_CHUNK,), jnp.float32),
        pltpu.VMEM((_CHUNK,), jnp.int32),
        pltpu.VMEM((_CHUNK,), jnp.int32),
        pltpu.VMEM((2 * _L,), jnp.float32),
        pltpu.SemaphoreType.DMA,
        pltpu.SemaphoreType.DMA,
    ],
    compiler_params=pltpu.CompilerParams(needs_layout_passes=False),
)(_sc_body)


def kernel(outputs, targets, thin_mask, thin_weight):
    d = outputs[:, 1] - outputs[:, 0]
    partials = _sc_kernel(d, targets, thin_mask)
    pr = partials.reshape(_NW, 2, _L)
    s_all = jnp.sum(pr[:, 0, :])
    s_thin = jnp.sum(pr[:, 1, :])
    tw = jnp.asarray(thin_weight, jnp.float32)
    loss = (s_all + (tw - 1.0) * s_thin) * (1.0 / _N)
    return loss.astype(jnp.float32)
